# baseline (speedup 1.0000x reference)
import functools

import jax
import jax.numpy as jnp
from jax import lax
from jax.experimental import pallas as pl
from jax.experimental.pallas import tpu as pltpu
from jax.experimental.pallas import tpu_sc as plsc

THR = 0.7
KEEP_MIN = 100000

B, C, H, W = 4, 19, 512, 512
ROWS = B * H
ROWS_SC = 768
ROWS_TC = ROWS - ROWS_SC
ROW0_SC = ROWS_TC
NW = 32
RPT = ROWS_SC // NW
CR, CW = 8, 256
NCW = W // CW
NCH = (RPT // CR) * NCW
L = 16


def _sc_body(score_hbm, tgt_hbm, se_hbm, st_hbm, sbuf, tbuf, sebuf, stbuf,
             insem, outsem):
    wid = lax.axis_index("s") * 2 + lax.axis_index("c")
    g0 = ROW0_SC + wid * RPT

    def chunk_off(j):
        g = g0 + (j // NCW) * CR
        return g // H, g % H, (j % NCW) * CW, g - ROW0_SC

    def in_copies(j, bank):
        b, h0, w0, _ = chunk_off(j)
        return (
            pltpu.make_async_copy(
                score_hbm.at[b, :, pl.ds(h0, CR), pl.ds(w0, CW)],
                sbuf.at[bank], insem.at[bank]),
            pltpu.make_async_copy(
                tgt_hbm.at[b, pl.ds(h0, CR), pl.ds(w0, CW)],
                tbuf.at[bank], insem.at[bank]),
        )

    def out_copies(j, bank):
        _, _, w0, gout = chunk_off(j)
        return (
            pltpu.make_async_copy(
                sebuf.at[bank], se_hbm.at[pl.ds(gout, CR), pl.ds(w0, CW)],
                outsem.at[bank]),
            pltpu.make_async_copy(
                stbuf.at[bank], st_hbm.at[pl.ds(gout, CR), pl.ds(w0, CW)],
                outsem.at[bank]),
        )

    for cp in in_copies(0, 0):
        cp.start()

    for j in range(NCH):
        bank = j % 2
        for cp in in_copies(j, bank):
            cp.wait()
        if j + 1 < NCH:
            for cp in in_copies(j + 1, 1 - bank):
                cp.start()
        if j >= 2:
            for cp in out_copies(j - 2, bank):
                cp.wait()

        def group(i, carry, bank=bank):
            r = i // (CW // L)
            sl = pl.ds((i % (CW // L)) * L, L)
            t = tbuf[bank, r, sl]
            se0 = jnp.zeros((L,), jnp.float32)
            se1 = jnp.zeros((L,), jnp.float32)
            st = jnp.zeros((L,), jnp.float32)
            for c in range(C):
                v = sbuf[bank, c, r, sl]
                if c % 2 == 0:
                    se0 = se0 + jnp.exp(v)
                else:
                    se1 = se1 + jnp.exp(v)
                st = jnp.where(t == c, v, st)
            sebuf[bank, r, sl] = se0 + se1
            stbuf[bank, r, sl] = st
            return carry

        lax.fori_loop(0, CR * CW // L, group, 0)
        for cp in out_copies(j, bank):
            cp.start()

    for cp in out_copies(NCH - 2, (NCH - 2) % 2):
        cp.wait()
    for cp in out_copies(NCH - 1, (NCH - 1) % 2):
        cp.wait()


def _sc_pass(score, target):
    mesh = plsc.VectorSubcoreMesh(core_axis_name="c", subcore_axis_name="s")
    fn = functools.partial(
        pl.kernel,
        mesh=mesh,
        out_type=[
            jax.ShapeDtypeStruct((ROWS_SC, W), jnp.float32),
            jax.ShapeDtypeStruct((ROWS_SC, W), jnp.float32),
        ],
        scratch_types=[
            pltpu.VMEM((2, C, CR, CW), jnp.float32),
            pltpu.VMEM((2, CR, CW), jnp.int32),
            pltpu.VMEM((2, CR, CW), jnp.float32),
            pltpu.VMEM((2, CR, CW), jnp.float32),
            pltpu.SemaphoreType.DMA((2,)),
            pltpu.SemaphoreType.DMA((2,)),
        ],
    )(_sc_body)
    return fn(score, target)


def _stats_init(acc_ref):
    acc_ref[0] = jnp.float32(0.0)
    acc_ref[1] = jnp.float32(0.0)
    acc_ref[2] = jnp.float32(0.0)


def _stats_accum(acc_ref, p, nll):
    lt = p < THR
    acc_ref[0] += jnp.sum(jnp.where(p <= THR, 1.0, 0.0))
    acc_ref[1] += jnp.sum(jnp.where(lt, 1.0, 0.0))
    acc_ref[2] += jnp.sum(jnp.where(lt, nll, 0.0))


RH = 128


def _tca_body(emit_pnll, score_ref, tgt_ref, *refs):
    if emit_pnll:
        p_ref, nll_ref, stats_ref, acc_ref = refs
    else:
        stats_ref, acc_ref = refs
    u = pl.program_id(0)

    @pl.when(u == 0)
    def _init():
        _stats_init(acc_ref)

    s = score_ref[0]
    t = tgt_ref[0]
    m = jnp.max(s, axis=0)
    e = jnp.exp(s - m[None])
    se = jnp.sum(e, axis=0)
    cls = lax.broadcasted_iota(jnp.int32, s.shape, 0)
    onehot = cls == t[None]
    e_t = jnp.sum(jnp.where(onehot, e, 0.0), axis=0)
    s_t = jnp.sum(jnp.where(onehot, s, 0.0), axis=0)
    p = e_t / se
    nll = jnp.log(se) - (s_t - m)
    if emit_pnll:
        p_ref[...] = p
        nll_ref[...] = nll
    _stats_accum(acc_ref, p, nll)

    @pl.when(u == pl.num_programs(0) - 1)
    def _fin():
        stats_ref[0] = acc_ref[0]
        stats_ref[1] = acc_ref[1]
        stats_ref[2] = acc_ref[2]


def _tc_pass_a(score, target, emit_pnll):
    upi = H // RH
    pnll_specs = [
        pl.BlockSpec((RH, W), lambda u: (u, 0)),
        pl.BlockSpec((RH, W), lambda u: (u, 0)),
    ]
    pnll_shapes = [
        jax.ShapeDtypeStruct((ROWS_TC, W), jnp.float32),
        jax.ShapeDtypeStruct((ROWS_TC, W), jnp.float32),
    ]
    return pl.pallas_call(
        functools.partial(_tca_body, emit_pnll),
        grid=(ROWS_TC // RH,),
        in_specs=[
            pl.BlockSpec((1, C, RH, W), lambda u: (u // upi, 0, u % upi, 0)),
            pl.BlockSpec((1, RH, W), lambda u: (u // upi, u % upi, 0)),
        ],
        out_specs=(pnll_specs if emit_pnll else []) + [
            pl.BlockSpec(memory_space=pltpu.SMEM, index_map=lambda u: (0,)),
        ],
        out_shape=(pnll_shapes if emit_pnll else []) + [
            jax.ShapeDtypeStruct((3,), jnp.float32),
        ],
        scratch_shapes=[pltpu.SMEM((3,), jnp.float32)],
    )(score, target)


def _tcb_body(emit_pnll, se_ref, st_ref, *refs):
    if emit_pnll:
        p_ref, nll_ref, stats_ref, acc_ref = refs
    else:
        stats_ref, acc_ref = refs
    i = pl.program_id(0)

    @pl.when(i == 0)
    def _init():
        _stats_init(acc_ref)

    se = se_ref[...]
    st = st_ref[...]
    p = jnp.exp(st) / se
    nll = jnp.log(se) - st
    if emit_pnll:
        p_ref[...] = p
        nll_ref[...] = nll
    _stats_accum(acc_ref, p, nll)

    @pl.when(i == pl.num_programs(0) - 1)
    def _fin():
        stats_ref[0] = acc_ref[0]
        stats_ref[1] = acc_ref[1]
        stats_ref[2] = acc_ref[2]


def _tc_pass_b(se, st, emit_pnll):
    RB = 256
    pnll_specs = [
        pl.BlockSpec((RB, W), lambda i: (i, 0)),
        pl.BlockSpec((RB, W), lambda i: (i, 0)),
    ]
    pnll_shapes = [
        jax.ShapeDtypeStruct((ROWS_SC, W), jnp.float32),
        jax.ShapeDtypeStruct((ROWS_SC, W), jnp.float32),
    ]
    return pl.pallas_call(
        functools.partial(_tcb_body, emit_pnll),
        grid=(ROWS_SC // RB,),
        in_specs=[
            pl.BlockSpec((RB, W), lambda i: (i, 0)),
            pl.BlockSpec((RB, W), lambda i: (i, 0)),
        ],
        out_specs=(pnll_specs if emit_pnll else []) + [
            pl.BlockSpec(memory_space=pltpu.SMEM, index_map=lambda i: (0,)),
        ],
        out_shape=(pnll_shapes if emit_pnll else []) + [
            jax.ShapeDtypeStruct((3,), jnp.float32),
        ],
        scratch_shapes=[pltpu.SMEM((3,), jnp.float32)],
    )(se, st)


def kernel(score, target):
    se_sc, st_sc = _sc_pass(score, target)
    stats_a = _tc_pass_a(score, target, emit_pnll=False)[0]
    stats_b = _tc_pass_b(se_sc, st_sc, emit_pnll=False)[0]
    stats = stats_a + stats_b
    cnt_le, cnt_lt, loss_sum = stats[0], stats[1], stats[2]

    def common(_):
        return loss_sum / cnt_lt

    def rare(_):
        p_a, nll_a, _ = _tc_pass_a(score, target, emit_pnll=True)
        p_b, nll_b, _ = _tc_pass_b(se_sc, st_sc, emit_pnll=True)
        p = jnp.concatenate([p_a, p_b], axis=0)
        nll = jnp.concatenate([nll_a, nll_b], axis=0)
        ps = jnp.sort(p.reshape(-1))
        thr = jnp.maximum(ps[KEEP_MIN - 1], jnp.float32(THR))
        keep = p < thr
        tot = jnp.sum(jnp.where(keep, nll, 0.0))
        cnt = jnp.sum(keep).astype(jnp.float32)
        return tot / cnt

    return lax.cond(cnt_le >= KEEP_MIN, common, rare, None)


# speedup vs baseline: 1.0517x; 1.0517x over previous
import functools
import math

import jax
import jax.numpy as jnp
from jax import lax
from jax.experimental import pallas as pl
from jax.experimental.pallas import tpu as pltpu
from jax.experimental.pallas import tpu_sc as plsc

THR = 0.7
LN_THR = math.log(0.7)
LN2 = math.log(2.0)
SQRT2 = math.sqrt(2.0)
LN_POLY = (-0.1423510541444221, 0.22399825665055528, -0.25522982057654203,
           0.33217087182475424, -0.4998020765878382, 1.0000156368635584,
           -1.0004320099919192e-06)
KEEP_MIN = 100000

B, C, H, W = 4, 19, 512, 512
ROWS = B * H
ROWS_SC = 768
ROWS_TC = ROWS - ROWS_SC
ROW0_SC = ROWS_TC
NW = 32
RPT = ROWS_SC // NW
CR, CW = 8, 256
NCW = W // CW
NCH = (RPT // CR) * NCW
L = 16


def _ln_approx(se):
    bits = lax.bitcast_convert_type(se, jnp.int32)
    ex = (bits >> 23) - 127
    man = (bits & 0x007FFFFF) | 0x3F800000
    f = lax.bitcast_convert_type(man, jnp.float32)
    big = f >= SQRT2
    f2 = jnp.where(big, 0.5 * f, f)
    exf = (ex + jnp.where(big, 1, 0)).astype(jnp.float32)
    t = f2 - 1.0
    p = jnp.full((L,), LN_POLY[0], jnp.float32)
    for coef in LN_POLY[1:]:
        p = p * t + coef
    return exf * LN2 + p


def _chunk_helpers(wid):
    g0 = ROW0_SC + wid * RPT

    def chunk_off(j):
        g = g0 + (j // NCW) * CR
        return g // H, g % H, (j % NCW) * CW, g - ROW0_SC

    return chunk_off


def _class_loop(sbuf, tbuf, bank, r, sl):
    t = tbuf[bank, r, sl]
    se0 = jnp.zeros((L,), jnp.float32)
    se1 = jnp.zeros((L,), jnp.float32)
    st = jnp.zeros((L,), jnp.float32)
    for c in range(C):
        v = sbuf[bank, c, r, sl]
        if c % 2 == 0:
            se0 = se0 + jnp.exp(v)
        else:
            se1 = se1 + jnp.exp(v)
        st = jnp.where(t == c, v, st)
    return se0 + se1, st


def _sc_stats_body(score_hbm, tgt_hbm, stats_hbm, sbuf, tbuf, statbuf,
                   insem, outsem):
    wid = lax.axis_index("s") * 2 + lax.axis_index("c")
    chunk_off = _chunk_helpers(wid)

    def in_copies(j, bank):
        b, h0, w0, _ = chunk_off(j)
        return (
            pltpu.make_async_copy(
                score_hbm.at[b, :, pl.ds(h0, CR), pl.ds(w0, CW)],
                sbuf.at[bank], insem.at[bank]),
            pltpu.make_async_copy(
                tgt_hbm.at[b, pl.ds(h0, CR), pl.ds(w0, CW)],
                tbuf.at[bank], insem.at[bank]),
        )

    for cp in in_copies(0, 0):
        cp.start()

    acc = (jnp.zeros((L,), jnp.float32), jnp.zeros((L,), jnp.float32),
           jnp.zeros((L,), jnp.float32))

    for j in range(NCH):
        bank = j % 2
        for cp in in_copies(j, bank):
            cp.wait()
        if j + 1 < NCH:
            for cp in in_copies(j + 1, 1 - bank):
                cp.start()

        def group(i, carry, bank=bank):
            a_le, a_lt, a_loss = carry
            r = i // (CW // L)
            sl = pl.ds((i % (CW // L)) * L, L)
            se, st = _class_loop(sbuf, tbuf, bank, r, sl)
            lnse = _ln_approx(se)
            lp = st - lnse
            nll = lnse - st
            le = lp <= LN_THR
            lt = lp < LN_THR
            a_le = a_le + jnp.where(le, 1.0, 0.0)
            a_lt = a_lt + jnp.where(lt, 1.0, 0.0)
            a_loss = a_loss + jnp.where(lt, nll, 0.0)
            return a_le, a_lt, a_loss

        acc = lax.fori_loop(0, CR * CW // L, group, acc)

    statbuf[0, pl.ds(0, L)] = acc[0]
    statbuf[1, pl.ds(0, L)] = acc[1]
    statbuf[2, pl.ds(0, L)] = acc[2]
    pltpu.make_async_copy(statbuf, stats_hbm.at[wid], outsem).start()
    pltpu.make_async_copy(statbuf, stats_hbm.at[wid], outsem).wait()


def _sc_stats_pass(score, target):
    mesh = plsc.VectorSubcoreMesh(core_axis_name="c", subcore_axis_name="s")
    fn = functools.partial(
        pl.kernel,
        mesh=mesh,
        out_type=jax.ShapeDtypeStruct((NW, 8, 128), jnp.float32),
        scratch_types=[
            pltpu.VMEM((2, C, CR, CW), jnp.float32),
            pltpu.VMEM((2, CR, CW), jnp.int32),
            pltpu.VMEM((8, 128), jnp.float32),
            pltpu.SemaphoreType.DMA((2,)),
            pltpu.SemaphoreType.DMA,
        ],
    )(_sc_stats_body)
    return fn(score, target)


def _sc_emit_body(score_hbm, tgt_hbm, se_hbm, st_hbm, sbuf, tbuf, sebuf,
                  stbuf, insem, outsem):
    wid = lax.axis_index("s") * 2 + lax.axis_index("c")
    chunk_off = _chunk_helpers(wid)

    def in_copies(j, bank):
        b, h0, w0, _ = chunk_off(j)
        return (
            pltpu.make_async_copy(
                score_hbm.at[b, :, pl.ds(h0, CR), pl.ds(w0, CW)],
                sbuf.at[bank], insem.at[bank]),
            pltpu.make_async_copy(
                tgt_hbm.at[b, pl.ds(h0, CR), pl.ds(w0, CW)],
                tbuf.at[bank], insem.at[bank]),
        )

    def out_copies(j, bank):
        _, _, w0, gout = chunk_off(j)
        return (
            pltpu.make_async_copy(
                sebuf.at[bank], se_hbm.at[pl.ds(gout, CR), pl.ds(w0, CW)],
                outsem.at[bank]),
            pltpu.make_async_copy(
                stbuf.at[bank], st_hbm.at[pl.ds(gout, CR), pl.ds(w0, CW)],
                outsem.at[bank]),
        )

    for cp in in_copies(0, 0):
        cp.start()

    for j in range(NCH):
        bank = j % 2
        for cp in in_copies(j, bank):
            cp.wait()
        if j + 1 < NCH:
            for cp in in_copies(j + 1, 1 - bank):
                cp.start()
        if j >= 2:
            for cp in out_copies(j - 2, bank):
                cp.wait()

        def group(i, carry, bank=bank):
            r = i // (CW // L)
            sl = pl.ds((i % (CW // L)) * L, L)
            se, st = _class_loop(sbuf, tbuf, bank, r, sl)
            sebuf[bank, r, sl] = se
            stbuf[bank, r, sl] = st
            return carry

        lax.fori_loop(0, CR * CW // L, group, 0)
        for cp in out_copies(j, bank):
            cp.start()

    for cp in out_copies(NCH - 2, (NCH - 2) % 2):
        cp.wait()
    for cp in out_copies(NCH - 1, (NCH - 1) % 2):
        cp.wait()


def _sc_emit_pass(score, target):
    mesh = plsc.VectorSubcoreMesh(core_axis_name="c", subcore_axis_name="s")
    fn = functools.partial(
        pl.kernel,
        mesh=mesh,
        out_type=[
            jax.ShapeDtypeStruct((ROWS_SC, W), jnp.float32),
            jax.ShapeDtypeStruct((ROWS_SC, W), jnp.float32),
        ],
        scratch_types=[
            pltpu.VMEM((2, C, CR, CW), jnp.float32),
            pltpu.VMEM((2, CR, CW), jnp.int32),
            pltpu.VMEM((2, CR, CW), jnp.float32),
            pltpu.VMEM((2, CR, CW), jnp.float32),
            pltpu.SemaphoreType.DMA((2,)),
            pltpu.SemaphoreType.DMA((2,)),
        ],
    )(_sc_emit_body)
    return fn(score, target)


def _stats_init(acc_ref):
    acc_ref[0] = jnp.float32(0.0)
    acc_ref[1] = jnp.float32(0.0)
    acc_ref[2] = jnp.float32(0.0)


def _stats_accum(acc_ref, p, nll):
    lt = p < THR
    acc_ref[0] += jnp.sum(jnp.where(p <= THR, 1.0, 0.0))
    acc_ref[1] += jnp.sum(jnp.where(lt, 1.0, 0.0))
    acc_ref[2] += jnp.sum(jnp.where(lt, nll, 0.0))


RH = 128


def _tca_body(emit_pnll, score_ref, tgt_ref, *refs):
    if emit_pnll:
        p_ref, nll_ref, stats_ref, acc_ref = refs
    else:
        stats_ref, acc_ref = refs
    u = pl.program_id(0)

    @pl.when(u == 0)
    def _init():
        _stats_init(acc_ref)

    s = score_ref[0]
    t = tgt_ref[0]
    m = jnp.max(s, axis=0)
    e = jnp.exp(s - m[None])
    se = jnp.sum(e, axis=0)
    cls = lax.broadcasted_iota(jnp.int32, s.shape, 0)
    onehot = cls == t[None]
    e_t = jnp.sum(jnp.where(onehot, e, 0.0), axis=0)
    s_t = jnp.sum(jnp.where(onehot, s, 0.0), axis=0)
    p = e_t / se
    nll = jnp.log(se) - (s_t - m)
    if emit_pnll:
        p_ref[...] = p
        nll_ref[...] = nll
    _stats_accum(acc_ref, p, nll)

    @pl.when(u == pl.num_programs(0) - 1)
    def _fin():
        stats_ref[0] = acc_ref[0]
        stats_ref[1] = acc_ref[1]
        stats_ref[2] = acc_ref[2]


def _tc_pass_a(score, target, emit_pnll):
    upi = H // RH
    pnll_specs = [
        pl.BlockSpec((RH, W), lambda u: (u, 0)),
        pl.BlockSpec((RH, W), lambda u: (u, 0)),
    ]
    pnll_shapes = [
        jax.ShapeDtypeStruct((ROWS_TC, W), jnp.float32),
        jax.ShapeDtypeStruct((ROWS_TC, W), jnp.float32),
    ]
    return pl.pallas_call(
        functools.partial(_tca_body, emit_pnll),
        grid=(ROWS_TC // RH,),
        in_specs=[
            pl.BlockSpec((1, C, RH, W), lambda u: (u // upi, 0, u % upi, 0)),
            pl.BlockSpec((1, RH, W), lambda u: (u // upi, u % upi, 0)),
        ],
        out_specs=(pnll_specs if emit_pnll else []) + [
            pl.BlockSpec(memory_space=pltpu.SMEM, index_map=lambda u: (0,)),
        ],
        out_shape=(pnll_shapes if emit_pnll else []) + [
            jax.ShapeDtypeStruct((3,), jnp.float32),
        ],
        scratch_shapes=[pltpu.SMEM((3,), jnp.float32)],
    )(score, target)


def _tcb_body(se_ref, st_ref, p_ref, nll_ref):
    se = se_ref[...]
    st = st_ref[...]
    p_ref[...] = jnp.exp(st) / se
    nll_ref[...] = jnp.log(se) - st


def _tc_pass_b(se, st):
    RB = 256
    return pl.pallas_call(
        _tcb_body,
        grid=(ROWS_SC // RB,),
        in_specs=[
            pl.BlockSpec((RB, W), lambda i: (i, 0)),
            pl.BlockSpec((RB, W), lambda i: (i, 0)),
        ],
        out_specs=[
            pl.BlockSpec((RB, W), lambda i: (i, 0)),
            pl.BlockSpec((RB, W), lambda i: (i, 0)),
        ],
        out_shape=[
            jax.ShapeDtypeStruct((ROWS_SC, W), jnp.float32),
            jax.ShapeDtypeStruct((ROWS_SC, W), jnp.float32),
        ],
    )(se, st)


def kernel(score, target):
    sc_stats = _sc_stats_pass(score, target)
    stats_a = _tc_pass_a(score, target, emit_pnll=False)[0]
    sc_sums = jnp.sum(sc_stats[:, :3, :L], axis=(0, 2))
    cnt_le = stats_a[0] + sc_sums[0]
    cnt_lt = stats_a[1] + sc_sums[1]
    loss_sum = stats_a[2] + sc_sums[2]

    def common(_):
        return loss_sum / cnt_lt

    def rare(_):
        p_a, nll_a, _ = _tc_pass_a(score, target, emit_pnll=True)
        se_sc, st_sc = _sc_emit_pass(score, target)
        p_b, nll_b = _tc_pass_b(se_sc, st_sc)
        p = jnp.concatenate([p_a, p_b], axis=0)
        nll = jnp.concatenate([nll_a, nll_b], axis=0)
        ps = jnp.sort(p.reshape(-1))
        thr = jnp.maximum(ps[KEEP_MIN - 1], jnp.float32(THR))
        keep = p < thr
        tot = jnp.sum(jnp.where(keep, nll, 0.0))
        cnt = jnp.sum(keep).astype(jnp.float32)
        return tot / cnt

    return lax.cond(cnt_le >= KEEP_MIN, common, rare, None)
